# Initial kernel scaffold; baseline (speedup 1.0000x reference)
import jax
import jax.numpy as jnp
from jax.experimental import pallas as pl


def kernel(h, edge_index, rel_types, enc_W, enc_b, dec_W, dec_b, c1_W, c1_Wself, c1_b, c2_W, c2_Wself, c2_b, cls_W, cls_b):
    raise NotImplementedError("write your pallas kernel here")


# speedup vs baseline: 18.5668x; 18.5668x over previous
import functools

import jax
import jax.numpy as jnp
from jax import lax
from jax.experimental import pallas as pl
from jax.experimental.pallas import tpu as pltpu
from jax.experimental.pallas import tpu_sc as plsc

N = 10000
E = 320000
D = 128
R = 8
C = 16

NC = 2
NS = 16
NW = NC * NS
LANES = 16

EPW = E // NW
K = 128
CH = -(-EPW // K)
PAD = CH * K - EPW
N2 = 10240
RPT = N2 // NS

BN = 1000


def _enc_body(h_ref, encW_ref, encb_ref, decW_ref, decb_ref,
              W_ref, Wself_ref, b_ref,
              he_ref, dec_ref, self_ref, proj_ref):
    h = h_ref[...]
    he = jnp.maximum(jnp.dot(h, encW_ref[...],
                             preferred_element_type=jnp.float32)
                     + encb_ref[...], 0.0)
    he_ref[...] = he
    dec_ref[...] = jnp.dot(he, decW_ref[...],
                           preferred_element_type=jnp.float32) + decb_ref[...]
    self_ref[...] = jnp.dot(he, Wself_ref[...],
                            preferred_element_type=jnp.float32) + b_ref[...]
    for r in range(R):
        proj_ref[r] = jnp.dot(he, W_ref[r], preferred_element_type=jnp.float32)


def _mid_body(p_ref, self1_ref, W_ref, Wself_ref, b_ref,
              self2_ref, proj_ref):
    x = jnp.maximum(p_ref[0] + p_ref[1] + self1_ref[...], 0.0)
    self2_ref[...] = jnp.dot(x, Wself_ref[...],
                             preferred_element_type=jnp.float32) + b_ref[...]
    for r in range(R):
        proj_ref[r] = jnp.dot(x, W_ref[r], preferred_element_type=jnp.float32)


def _head_body(p_ref, self2_ref, clsW_ref, clsb_ref, probs_ref, acc_ref):
    i = pl.program_id(0)
    y = jnp.maximum(p_ref[0] + p_ref[1] + self2_ref[...], 0.0)
    part = jnp.sum(y, axis=0, keepdims=True)

    @pl.when(i == 0)
    def _():
        acc_ref[...] = part

    @pl.when(i > 0)
    def _():
        acc_ref[...] = acc_ref[...] + part

    @pl.when(i == pl.num_programs(0) - 1)
    def _():
        hg = acc_ref[...] * (1.0 / N)
        logits = jnp.dot(hg, clsW_ref[...],
                         preferred_element_type=jnp.float32) + clsb_ref[...]
        m = jnp.max(logits, axis=1, keepdims=True)
        e = jnp.exp(logits - m)
        probs_ref[...] = e / jnp.sum(e, axis=1, keepdims=True)


def _row_spec(bn):
    return pl.BlockSpec((bn, D), lambda i: (i, 0))


def _full_spec(shape):
    nd = len(shape)
    return pl.BlockSpec(shape, lambda i: (0,) * nd)


_sc_mesh = plsc.VectorSubcoreMesh(core_axis_name="c", subcore_axis_name="s")


@functools.partial(
    pl.kernel,
    mesh=_sc_mesh,
    out_type=jax.ShapeDtypeStruct((NC, N2, D), jnp.float32),
    scratch_types=[
        pltpu.VMEM((CH, K), jnp.int32),
        pltpu.VMEM((CH, K), jnp.int32),
        pltpu.VMEM((K, D), jnp.float32),
        pltpu.VMEM_SHARED((N2, D), jnp.float32),
        pltpu.SemaphoreType.DMA,
    ],
)
def _edge_agg(proj_hbm, gidx_hbm, didx_hbm, out_hbm,
              gidx_v, didx_v, rows_v, agg, sem):
    c = lax.axis_index("c")
    s = lax.axis_index("s")
    w = c * NS + s

    def zero_row(i, carry):
        for jj in range(D // LANES):
            rows_v[i, pl.ds(jj * LANES, LANES)] = jnp.zeros((LANES,),
                                                            jnp.float32)
        return carry

    lax.fori_loop(0, K, zero_row, 0)
    base = s * RPT
    for q in range(RPT // K):
        pltpu.sync_copy(rows_v, agg.at[pl.ds(base + q * K, K)])
    plsc.subcore_barrier()

    pltpu.sync_copy(gidx_hbm.at[w], gidx_v)
    pltpu.sync_copy(didx_hbm.at[w], didx_v)

    def chunk(j, carry):
        pltpu.async_copy(proj_hbm.at[gidx_v.at[j]], rows_v, sem).wait()
        pltpu.sync_copy(rows_v, agg.at[didx_v.at[j]], add=True)
        return carry

    lax.fori_loop(0, CH, chunk, 0)
    plsc.subcore_barrier()

    pltpu.sync_copy(agg.at[pl.ds(base, RPT)], out_hbm.at[c, pl.ds(base, RPT)])


@jax.jit
def kernel(h, edge_index, rel_types, enc_W, enc_b, dec_W, dec_b,
           c1_W, c1_Wself, c1_b, c2_W, c2_Wself, c2_b, cls_W, cls_b):
    h = jnp.asarray(h, jnp.float32)
    enc_b2 = enc_b.reshape(1, D)
    dec_b2 = dec_b.reshape(1, D)
    c1_b2 = c1_b.reshape(1, D)
    c2_b2 = c2_b.reshape(1, D)
    cls_b2 = cls_b.reshape(1, C)

    src = edge_index[0]
    dst = edge_index[1]
    gidx = (rel_types * N + src).astype(jnp.int32).reshape(NW, EPW)
    didx = dst.astype(jnp.int32).reshape(NW, EPW)
    gidx = jnp.pad(gidx, ((0, 0), (0, PAD))).reshape(NW, CH, K)
    didx = jnp.pad(didx, ((0, 0), (0, PAD)),
                   constant_values=N).reshape(NW, CH, K)

    grid = N // BN

    he, decoded, self1, proj1 = pl.pallas_call(
        _enc_body,
        grid=(grid,),
        in_specs=[_row_spec(BN), _full_spec((D, D)), _full_spec((1, D)),
                  _full_spec((D, D)), _full_spec((1, D)),
                  _full_spec((R, D, D)), _full_spec((D, D)),
                  _full_spec((1, D))],
        out_specs=[_row_spec(BN), _row_spec(BN), _row_spec(BN),
                   pl.BlockSpec((R, BN, D), lambda i: (0, i, 0))],
        out_shape=[jax.ShapeDtypeStruct((N, D), jnp.float32),
                   jax.ShapeDtypeStruct((N, D), jnp.float32),
                   jax.ShapeDtypeStruct((N, D), jnp.float32),
                   jax.ShapeDtypeStruct((R, N, D), jnp.float32)],
    )(h, enc_W, enc_b2, dec_W, dec_b2, c1_W, c1_Wself, c1_b2)

    part1 = _edge_agg(proj1.reshape(R * N, D), gidx, didx)

    self2, proj2 = pl.pallas_call(
        _mid_body,
        grid=(grid,),
        in_specs=[pl.BlockSpec((NC, BN, D), lambda i: (0, i, 0)),
                  _row_spec(BN), _full_spec((R, D, D)), _full_spec((D, D)),
                  _full_spec((1, D))],
        out_specs=[_row_spec(BN),
                   pl.BlockSpec((R, BN, D), lambda i: (0, i, 0))],
        out_shape=[jax.ShapeDtypeStruct((N, D), jnp.float32),
                   jax.ShapeDtypeStruct((R, N, D), jnp.float32)],
    )(part1, self1, c2_W, c2_Wself, c2_b2)

    part2 = _edge_agg(proj2.reshape(R * N, D), gidx, didx)

    probs = pl.pallas_call(
        _head_body,
        grid=(grid,),
        in_specs=[pl.BlockSpec((NC, BN, D), lambda i: (0, i, 0)),
                  _row_spec(BN), _full_spec((D, C)), _full_spec((1, C))],
        out_specs=pl.BlockSpec((1, C), lambda i: (0, 0)),
        out_shape=jax.ShapeDtypeStruct((1, C), jnp.float32),
        scratch_shapes=[pltpu.VMEM((1, D), jnp.float32)],
    )(part2, self2, cls_W, cls_b2)

    return (decoded, probs)
